# Initial kernel scaffold; baseline (speedup 1.0000x reference)
import jax
import jax.numpy as jnp
from jax.experimental import pallas as pl


def kernel(x, edge_index, W1, b1, W2, b2):
    raise NotImplementedError("write your pallas kernel here")


# speedup vs baseline: 9.3774x; 9.3774x over previous
import functools

import jax
import jax.numpy as jnp
from jax import lax
from jax.experimental import pallas as pl
from jax.experimental.pallas import tpu as pltpu
from jax.experimental.pallas import tpu_sc as plsc

NC = 2
NS = 16
LANES = 16


def _rsqrt16(v):
    i = plsc.bitcast(v, jnp.int32)
    i = jnp.int32(0x5F3759DF) - lax.shift_right_logical(i, 1)
    y = plsc.bitcast(i, jnp.float32)
    for _ in range(3):
        y = y * (1.5 - 0.5 * v * y * y)
    return y


def _mlp_tc(x, W1, b1, W2s, b2s, DC):
    N, D_IN = x.shape
    D_HID = W1.shape[0]
    BN = 1000

    def body(x_ref, w1_ref, b1_ref, w2_ref, b2_ref, o_ref):
        h1 = lax.dot_general(x_ref[...], w1_ref[...],
                             (((1,), (1,)), ((), ())),
                             preferred_element_type=jnp.float32)
        h1 = jnp.maximum(h1 + b1_ref[...], 0.0)
        for cc in range(NC):
            o_ref[cc] = lax.dot_general(h1, w2_ref[cc],
                                        (((1,), (1,)), ((), ())),
                                        preferred_element_type=jnp.float32) + b2_ref[cc]

    return pl.pallas_call(
        body,
        grid=(N // BN,),
        in_specs=[
            pl.BlockSpec((BN, D_IN), lambda i: (i, 0)),
            pl.BlockSpec((D_HID, D_IN), lambda i: (0, 0)),
            pl.BlockSpec((1, D_HID), lambda i: (0, 0)),
            pl.BlockSpec((NC, DC, D_HID), lambda i: (0, 0, 0)),
            pl.BlockSpec((NC, 1, DC), lambda i: (0, 0, 0)),
        ],
        out_specs=pl.BlockSpec((NC, BN, DC), lambda i: (0, i, 0)),
        out_shape=jax.ShapeDtypeStruct((NC, N, DC), jnp.float32),
    )(x, W1, b1.reshape(1, D_HID), W2s, b2s)


def _appnp_sc(h2, src3, dst3, *, N, DC, K, alpha):
    _, NB, B = src3.shape
    RT = N // NS
    RCH = 125
    NQ = RT // RCH
    keep = 1.0 - alpha

    mesh = plsc.VectorSubcoreMesh(core_axis_name="c", subcore_axis_name="s")

    @functools.partial(
        pl.kernel,
        out_type=jax.ShapeDtypeStruct((NC, N, DC), jnp.float32),
        mesh=mesh,
        compiler_params=pltpu.CompilerParams(use_tc_tiling_on_sc=False,
                                             needs_layout_passes=False),
        scratch_types=[
            pltpu.VMEM_SHARED((N, DC), jnp.float32),
            pltpu.VMEM_SHARED((N, DC), jnp.float32),
            pltpu.VMEM_SHARED((N, LANES), jnp.float32),
            pltpu.VMEM((2, B), jnp.int32),
            pltpu.VMEM((2, B), jnp.int32),
            pltpu.VMEM((RCH, LANES), jnp.float32),
            pltpu.VMEM((B, LANES), jnp.float32),
            pltpu.VMEM((B, DC), jnp.float32),
            pltpu.VMEM((B, DC), jnp.float32),
            pltpu.VMEM((RT, LANES), jnp.float32),
            pltpu.SMEM((RT,), jnp.float32),
            pltpu.SMEM((RT,), jnp.float32),
        ],
    )
    def k(h_hbm, src_hbm, dst_hbm, out_hbm,
          u_sh, agg_sh, deg_sh, src_b, dst_b, degc_v, ones_v, buf0, buf1,
          sd_v, d2_s, a2_s):
        c = lax.axis_index("c")
        s = lax.axis_index("s")
        r0 = s * RT

        def ones_body(i, carry):
            ones_v[i, :] = jnp.full((LANES,), 1.0, jnp.float32)
            return carry
        lax.fori_loop(0, B, ones_body, 0)

        for q in range(NQ):
            pltpu.sync_copy(ones_v.at[pl.ds(0, RCH)],
                            deg_sh.at[pl.ds(r0 + q * RCH, RCH)])
        plsc.subcore_barrier()

        def deg_body(j, carry):
            pltpu.sync_copy(dst_hbm.at[s, j], dst_b.at[0])
            pltpu.sync_copy(ones_v, deg_sh.at[dst_b.at[0]], add=True)
            return carry
        lax.fori_loop(0, NB, deg_body, 0)
        plsc.subcore_barrier()

        for q in range(NQ):
            rows = pl.ds(r0 + q * RCH, RCH)
            pltpu.sync_copy(deg_sh.at[rows], degc_v)
            pltpu.sync_copy(h_hbm.at[c, rows], buf0.at[pl.ds(0, RCH)])

            def prow(rr, carry):
                r = q * RCH + rr
                dg = degc_v[rr, :]
                y = _rsqrt16(dg)
                d2_s[r] = jnp.max(keep / dg)
                a2_s[r] = jnp.max(alpha * y)
                sd_v[r, :] = dg * y
                for g in range(DC // LANES):
                    sl = pl.ds(LANES * g, LANES)
                    buf1[rr, sl] = y * buf0[rr, sl]
                return carry
            lax.fori_loop(0, RCH, prow, 0)
            pltpu.sync_copy(buf1.at[pl.ds(0, RCH)], u_sh.at[rows])
            pltpu.sync_copy(buf1.at[pl.ds(0, RCH)], agg_sh.at[rows])
        plsc.subcore_barrier()

        for _t in range(K):
            def ebody(j, carry):
                pltpu.sync_copy(src_hbm.at[s, j], src_b.at[0])
                pltpu.sync_copy(dst_hbm.at[s, j], dst_b.at[0])
                pltpu.sync_copy(u_sh.at[src_b.at[0]], buf0)
                pltpu.sync_copy(buf0, agg_sh.at[dst_b.at[0]], add=True)
                return carry
            lax.fori_loop(0, NB, ebody, 0)
            plsc.subcore_barrier()

            for q in range(NQ):
                rows = pl.ds(r0 + q * RCH, RCH)
                pltpu.sync_copy(agg_sh.at[rows], buf1.at[pl.ds(0, RCH)])
                pltpu.sync_copy(h_hbm.at[c, rows], buf0.at[pl.ds(0, RCH)])

                def drow(rr, carry):
                    r = q * RCH + rr
                    d2 = jnp.full((LANES,), d2_s[r], jnp.float32)
                    a2 = jnp.full((LANES,), a2_s[r], jnp.float32)
                    for g in range(DC // LANES):
                        sl = pl.ds(LANES * g, LANES)
                        buf1[rr, sl] = d2 * buf1[rr, sl] + a2 * buf0[rr, sl]
                    return carry
                lax.fori_loop(0, RCH, drow, 0)
                pltpu.sync_copy(buf1.at[pl.ds(0, RCH)], u_sh.at[rows])
                pltpu.sync_copy(buf1.at[pl.ds(0, RCH)], agg_sh.at[rows])
            plsc.subcore_barrier()

        for q in range(NQ):
            rows = pl.ds(r0 + q * RCH, RCH)
            pltpu.sync_copy(u_sh.at[rows], buf1.at[pl.ds(0, RCH)])

            def orow(rr, carry):
                r = q * RCH + rr
                sd = sd_v[r, :]
                for g in range(DC // LANES):
                    sl = pl.ds(LANES * g, LANES)
                    buf1[rr, sl] = sd * buf1[rr, sl]
                return carry
            lax.fori_loop(0, RCH, orow, 0)
            pltpu.sync_copy(buf1.at[pl.ds(0, RCH)], out_hbm.at[c, rows])

    return k(h2, src3, dst3)


def kernel(x, edge_index, W1, b1, W2, b2):
    N, _ = x.shape
    D_OUT = W2.shape[0]
    E = edge_index.shape[1]
    DC = D_OUT // NC
    K = 10
    alpha = 0.1

    ET = E // NS
    B = 160
    NB = ET // B

    W2s = W2.reshape(NC, DC, W2.shape[1])
    b2s = b2.reshape(NC, 1, DC)
    h2 = _mlp_tc(x, W1, b1, W2s, b2s, DC)

    src3 = edge_index[0].reshape(NS, NB, B)
    dst3 = edge_index[1].reshape(NS, NB, B)

    out2 = _appnp_sc(h2, src3, dst3, N=N, DC=DC, K=K, alpha=alpha)
    return jnp.concatenate([out2[0], out2[1]], axis=1)


# speedup vs baseline: 15.7795x; 1.6827x over previous
import functools

import jax
import jax.numpy as jnp
from jax import lax
from jax.experimental import pallas as pl
from jax.experimental.pallas import tpu as pltpu
from jax.experimental.pallas import tpu_sc as plsc

NC = 2
NS = 16
LANES = 16


def _rsqrt16(v):
    i = plsc.bitcast(v, jnp.int32)
    i = jnp.int32(0x5F3759DF) - lax.shift_right_logical(i, 1)
    y = plsc.bitcast(i, jnp.float32)
    for _ in range(3):
        y = y * (1.5 - 0.5 * v * y * y)
    return y


def _mlp_tc(x, W1, b1, W2s, b2s, DC):
    N, D_IN = x.shape
    D_HID = W1.shape[0]
    BN = 1000

    def body(x_ref, w1_ref, b1_ref, w2_ref, b2_ref, o_ref):
        h1 = lax.dot_general(x_ref[...], w1_ref[...],
                             (((1,), (1,)), ((), ())),
                             preferred_element_type=jnp.float32)
        h1 = jnp.maximum(h1 + b1_ref[...], 0.0)
        for cc in range(NC):
            o_ref[cc] = lax.dot_general(h1, w2_ref[cc],
                                        (((1,), (1,)), ((), ())),
                                        preferred_element_type=jnp.float32) + b2_ref[cc]

    return pl.pallas_call(
        body,
        grid=(N // BN,),
        in_specs=[
            pl.BlockSpec((BN, D_IN), lambda i: (i, 0)),
            pl.BlockSpec((D_HID, D_IN), lambda i: (0, 0)),
            pl.BlockSpec((1, D_HID), lambda i: (0, 0)),
            pl.BlockSpec((NC, DC, D_HID), lambda i: (0, 0, 0)),
            pl.BlockSpec((NC, 1, DC), lambda i: (0, 0, 0)),
        ],
        out_specs=pl.BlockSpec((NC, BN, DC), lambda i: (0, i, 0)),
        out_shape=jax.ShapeDtypeStruct((NC, N, DC), jnp.float32),
    )(x, W1, b1.reshape(1, D_HID), W2s, b2s)


def _appnp_sc(h2, src4, dst3, *, N, DC, K, alpha):
    _, _, NB, B = src4.shape
    RT = N // NS
    RCH = 125
    NQ = RT // RCH
    keep = 1.0 - alpha

    mesh = plsc.VectorSubcoreMesh(core_axis_name="c", subcore_axis_name="s")

    @functools.partial(
        pl.kernel,
        out_type=jax.ShapeDtypeStruct((NC * N, DC), jnp.float32),
        mesh=mesh,
        compiler_params=pltpu.CompilerParams(use_tc_tiling_on_sc=False,
                                             needs_layout_passes=False),
        scratch_types=[
            pltpu.VMEM_SHARED((N, DC), jnp.float32),
            pltpu.VMEM((NB, B), jnp.int32),
            pltpu.VMEM((NB, B), jnp.int32),
            pltpu.VMEM((2, B, DC), jnp.float32),
            pltpu.VMEM((B, DC), jnp.float32),
            pltpu.VMEM((RT, LANES), jnp.float32),
            pltpu.SMEM((RT,), jnp.float32),
            pltpu.SMEM((RT,), jnp.float32),
            pltpu.SemaphoreType.DMA,
            pltpu.SemaphoreType.DMA,
        ],
    )
    def k(h_hbm, src_hbm, dst_hbm, u_hbm,
          agg_sh, src_v, dst_v, rbuf, buf1, sd_v, d2_s, a2_s, gsem, ssem):
        c = lax.axis_index("c")
        s = lax.axis_index("s")
        r0 = s * RT
        u0r = c * N + r0

        pltpu.sync_copy(src_hbm.at[c, s], src_v)
        pltpu.sync_copy(dst_hbm.at[s], dst_v)

        def gather_start(j, p):
            pltpu.async_copy(u_hbm.at[src_v.at[j]], rbuf.at[p], gsem)

        def gather_wait():
            pltpu.make_async_copy(u_hbm.at[src_v.at[0]], rbuf.at[0], gsem).wait()

        def scatter_start(j, p):
            pltpu.async_copy(rbuf.at[p], agg_sh.at[dst_v.at[j]], ssem, add=True)

        def scatter_wait():
            pltpu.make_async_copy(rbuf.at[0], agg_sh.at[dst_v.at[0]],
                                  ssem).wait()

        def ones_body(i, carry):
            for g in range(DC // LANES):
                rbuf[0, i, pl.ds(LANES * g, LANES)] = jnp.full(
                    (LANES,), 1.0, jnp.float32)
            return carry
        lax.fori_loop(0, B, ones_body, 0)

        for q in range(NQ):
            pltpu.sync_copy(rbuf.at[0, pl.ds(0, RCH)],
                            agg_sh.at[pl.ds(r0 + q * RCH, RCH)])
        plsc.subcore_barrier()

        def deg_body(j, carry):
            pltpu.sync_copy(rbuf.at[0], agg_sh.at[dst_v.at[j]], add=True)
            return carry
        lax.fori_loop(0, NB, deg_body, 0)
        plsc.subcore_barrier()

        for q in range(NQ):
            rows = pl.ds(r0 + q * RCH, RCH)
            pltpu.sync_copy(agg_sh.at[rows], buf1.at[pl.ds(0, RCH)])
            pltpu.sync_copy(h_hbm.at[c, rows], rbuf.at[0, pl.ds(0, RCH)])

            def prow(rr, carry):
                r = q * RCH + rr
                dg = buf1[rr, pl.ds(0, LANES)]
                y = _rsqrt16(dg)
                d2_s[r] = jnp.max(keep / dg)
                a2_s[r] = jnp.max(alpha * y)
                sd_v[r, :] = dg * y
                for g in range(DC // LANES):
                    sl = pl.ds(LANES * g, LANES)
                    buf1[rr, sl] = y * rbuf[0, rr, sl]
                return carry
            lax.fori_loop(0, RCH, prow, 0)
            pltpu.sync_copy(buf1.at[pl.ds(0, RCH)],
                            u_hbm.at[pl.ds(u0r + q * RCH, RCH)])
            pltpu.sync_copy(buf1.at[pl.ds(0, RCH)], agg_sh.at[rows])
        plsc.subcore_barrier()

        for _t in range(K):
            gather_start(0, 0)

            def ebody(jj, carry):
                j0 = 2 * jj
                gather_wait()

                @pl.when(jj > 0)
                def _():
                    scatter_wait()
                gather_start(j0 + 1, 1)
                scatter_start(j0, 0)
                gather_wait()
                scatter_wait()
                gather_start(j0 + 2, 0)
                scatter_start(j0 + 1, 1)
                return carry
            lax.fori_loop(0, (NB - 1) // 2, ebody, 0)
            gather_wait()
            scatter_wait()
            scatter_start(NB - 1, 0)
            scatter_wait()
            plsc.subcore_barrier()

            for q in range(NQ):
                rows = pl.ds(r0 + q * RCH, RCH)
                pltpu.sync_copy(agg_sh.at[rows], buf1.at[pl.ds(0, RCH)])
                pltpu.sync_copy(h_hbm.at[c, rows], rbuf.at[0, pl.ds(0, RCH)])

                def drow(rr, carry):
                    r = q * RCH + rr
                    d2 = jnp.full((LANES,), d2_s[r], jnp.float32)
                    a2 = jnp.full((LANES,), a2_s[r], jnp.float32)
                    for g in range(DC // LANES):
                        sl = pl.ds(LANES * g, LANES)
                        buf1[rr, sl] = d2 * buf1[rr, sl] + a2 * rbuf[0, rr, sl]
                    return carry
                lax.fori_loop(0, RCH, drow, 0)
                pltpu.sync_copy(buf1.at[pl.ds(0, RCH)],
                                u_hbm.at[pl.ds(u0r + q * RCH, RCH)])
                pltpu.sync_copy(buf1.at[pl.ds(0, RCH)], agg_sh.at[rows])
            plsc.subcore_barrier()

        for q in range(NQ):
            urows = pl.ds(u0r + q * RCH, RCH)
            pltpu.sync_copy(u_hbm.at[urows], buf1.at[pl.ds(0, RCH)])

            def orow(rr, carry):
                r = q * RCH + rr
                sd = sd_v[r, :]
                for g in range(DC // LANES):
                    sl = pl.ds(LANES * g, LANES)
                    buf1[rr, sl] = sd * buf1[rr, sl]
                return carry
            lax.fori_loop(0, RCH, orow, 0)
            pltpu.sync_copy(buf1.at[pl.ds(0, RCH)], u_hbm.at[urows])

    return k(h2, src4, dst3)


def kernel(x, edge_index, W1, b1, W2, b2):
    N, _ = x.shape
    D_OUT = W2.shape[0]
    E = edge_index.shape[1]
    DC = D_OUT // NC
    K = 10
    alpha = 0.1

    ET = E // NS
    B = 160
    NB = ET // B

    W2s = W2.reshape(NC, DC, W2.shape[1])
    b2s = b2.reshape(NC, 1, DC)
    h2 = _mlp_tc(x, W1, b1, W2s, b2s, DC)

    src3 = edge_index[0].reshape(NS, NB, B)
    dst3 = edge_index[1].reshape(NS, NB, B)
    src4 = jnp.stack([src3, src3 + N])

    u = _appnp_sc(h2, src4, dst3, N=N, DC=DC, K=K, alpha=alpha)
    return jnp.concatenate([u[:N], u[N:]], axis=1)


# speedup vs baseline: 15.7808x; 1.0001x over previous
import functools

import jax
import jax.numpy as jnp
from jax import lax
from jax.experimental import pallas as pl
from jax.experimental.pallas import tpu as pltpu
from jax.experimental.pallas import tpu_sc as plsc

NC = 2
NS = 16
LANES = 16


def _rsqrt16(v):
    i = plsc.bitcast(v, jnp.int32)
    i = jnp.int32(0x5F3759DF) - lax.shift_right_logical(i, 1)
    y = plsc.bitcast(i, jnp.float32)
    for _ in range(3):
        y = y * (1.5 - 0.5 * v * y * y)
    return y


def _mlp_tc(x, W1, b1, W2s, b2s, DC):
    N, D_IN = x.shape
    D_HID = W1.shape[0]
    BN = 1000

    def body(x_ref, w1_ref, b1_ref, w2_ref, b2_ref, o_ref):
        h1 = lax.dot_general(x_ref[...], w1_ref[...],
                             (((1,), (1,)), ((), ())),
                             preferred_element_type=jnp.float32)
        h1 = jnp.maximum(h1 + b1_ref[...], 0.0)
        for cc in range(NC):
            o_ref[cc] = lax.dot_general(h1, w2_ref[cc],
                                        (((1,), (1,)), ((), ())),
                                        preferred_element_type=jnp.float32) + b2_ref[cc]

    return pl.pallas_call(
        body,
        grid=(N // BN,),
        in_specs=[
            pl.BlockSpec((BN, D_IN), lambda i: (i, 0)),
            pl.BlockSpec((D_HID, D_IN), lambda i: (0, 0)),
            pl.BlockSpec((1, D_HID), lambda i: (0, 0)),
            pl.BlockSpec((NC, DC, D_HID), lambda i: (0, 0, 0)),
            pl.BlockSpec((NC, 1, DC), lambda i: (0, 0, 0)),
        ],
        out_specs=pl.BlockSpec((NC, BN, DC), lambda i: (0, i, 0)),
        out_shape=jax.ShapeDtypeStruct((NC, N, DC), jnp.float32),
    )(x, W1, b1.reshape(1, D_HID), W2s, b2s)


def _appnp_sc(h2, src4, dst3, *, N, DC, K, alpha):
    _, _, NB, B = src4.shape
    RT = N // NS
    RCH = 125
    NQ = RT // RCH
    keep = 1.0 - alpha

    mesh = plsc.VectorSubcoreMesh(core_axis_name="c", subcore_axis_name="s")

    @functools.partial(
        pl.kernel,
        out_type=jax.ShapeDtypeStruct((NC * N, DC), jnp.float32),
        mesh=mesh,
        compiler_params=pltpu.CompilerParams(use_tc_tiling_on_sc=False,
                                             needs_layout_passes=False),
        scratch_types=[
            pltpu.VMEM_SHARED((N, DC), jnp.float32),
            pltpu.VMEM((NB, B), jnp.int32),
            pltpu.VMEM((NB, B), jnp.int32),
            pltpu.VMEM((2, B, DC), jnp.float32),
            pltpu.VMEM((B, DC), jnp.float32),
            pltpu.VMEM((RT, LANES), jnp.float32),
            pltpu.SMEM((RT,), jnp.float32),
            pltpu.SMEM((RT,), jnp.float32),
            pltpu.SemaphoreType.DMA,
            pltpu.SemaphoreType.DMA,
        ],
    )
    def k(h_hbm, src_hbm, dst_hbm, u_hbm,
          agg_sh, src_v, dst_v, rbuf, buf1, sd_v, d2_s, a2_s, gsem, ssem):
        c = lax.axis_index("c")
        s = lax.axis_index("s")
        r0 = s * RT
        u0r = c * N + r0

        pltpu.sync_copy(src_hbm.at[c, s], src_v)
        pltpu.sync_copy(dst_hbm.at[s], dst_v)

        def gather_start(j, p):
            pltpu.async_copy(u_hbm.at[src_v.at[j]], rbuf.at[p], gsem)

        def gather_wait():
            pltpu.make_async_copy(u_hbm.at[src_v.at[0]], rbuf.at[0], gsem).wait()

        def scatter_start(j, p):
            pltpu.async_copy(rbuf.at[p], agg_sh.at[dst_v.at[j]], ssem, add=True)

        def scatter_wait():
            pltpu.make_async_copy(rbuf.at[0], agg_sh.at[dst_v.at[0]],
                                  ssem).wait()

        def ones_body(i, carry):
            for g in range(DC // LANES):
                rbuf[0, i, pl.ds(LANES * g, LANES)] = jnp.full(
                    (LANES,), 1.0, jnp.float32)
            return carry
        lax.fori_loop(0, B, ones_body, 0)

        for q in range(NQ):
            pltpu.sync_copy(rbuf.at[0, pl.ds(0, RCH)],
                            agg_sh.at[pl.ds(r0 + q * RCH, RCH)])
        plsc.subcore_barrier()

        def deg_body(j, carry):
            pltpu.sync_copy(rbuf.at[0], agg_sh.at[dst_v.at[j]], add=True)
            return carry
        lax.fori_loop(0, NB, deg_body, 0)
        plsc.subcore_barrier()

        for q in range(NQ):
            rows = pl.ds(r0 + q * RCH, RCH)
            pltpu.sync_copy(agg_sh.at[rows], buf1.at[pl.ds(0, RCH)])
            pltpu.sync_copy(h_hbm.at[c, rows], rbuf.at[0, pl.ds(0, RCH)])

            def prow(rr, carry):
                r = q * RCH + rr
                dg = buf1[rr, pl.ds(0, LANES)]
                y = _rsqrt16(dg)
                d2_s[r] = jnp.max(keep / dg)
                a2_s[r] = jnp.max(alpha * y)
                sd_v[r, :] = dg * y
                for g in range(DC // LANES):
                    sl = pl.ds(LANES * g, LANES)
                    buf1[rr, sl] = y * rbuf[0, rr, sl]
                return carry
            lax.fori_loop(0, RCH, prow, 0)
            pltpu.sync_copy(buf1.at[pl.ds(0, RCH)],
                            u_hbm.at[pl.ds(u0r + q * RCH, RCH)])
            pltpu.sync_copy(buf1.at[pl.ds(0, RCH)], agg_sh.at[rows])
        plsc.subcore_barrier()

        for _t in range(K):
            with jax.named_scope("edge_phase"):
                gather_start(0, 0)

                def ebody(jj, carry):
                    j0 = 2 * jj
                    gather_wait()

                    @pl.when(jj > 0)
                    def _():
                        scatter_wait()
                    gather_start(j0 + 1, 1)
                    scatter_start(j0, 0)
                    gather_wait()
                    scatter_wait()
                    gather_start(j0 + 2, 0)
                    scatter_start(j0 + 1, 1)
                    return carry
                lax.fori_loop(0, (NB - 1) // 2, ebody, 0)
                gather_wait()
                scatter_wait()
                scatter_start(NB - 1, 0)
                scatter_wait()
                plsc.subcore_barrier()

            with jax.named_scope("dense_phase"):
                for q in range(NQ):
                    rows = pl.ds(r0 + q * RCH, RCH)
                    pltpu.sync_copy(agg_sh.at[rows], buf1.at[pl.ds(0, RCH)])
                    pltpu.sync_copy(h_hbm.at[c, rows], rbuf.at[0, pl.ds(0, RCH)])

                    def drow(rr, carry):
                        r = q * RCH + rr
                        d2 = jnp.full((LANES,), d2_s[r], jnp.float32)
                        a2 = jnp.full((LANES,), a2_s[r], jnp.float32)
                        for g in range(DC // LANES):
                            sl = pl.ds(LANES * g, LANES)
                            buf1[rr, sl] = d2 * buf1[rr, sl] + a2 * rbuf[0, rr, sl]
                        return carry
                    lax.fori_loop(0, RCH, drow, 0)
                    pltpu.sync_copy(buf1.at[pl.ds(0, RCH)],
                                    u_hbm.at[pl.ds(u0r + q * RCH, RCH)])
                    pltpu.sync_copy(buf1.at[pl.ds(0, RCH)], agg_sh.at[rows])
                plsc.subcore_barrier()

        for q in range(NQ):
            urows = pl.ds(u0r + q * RCH, RCH)
            pltpu.sync_copy(u_hbm.at[urows], buf1.at[pl.ds(0, RCH)])

            def orow(rr, carry):
                r = q * RCH + rr
                sd = sd_v[r, :]
                for g in range(DC // LANES):
                    sl = pl.ds(LANES * g, LANES)
                    buf1[rr, sl] = sd * buf1[rr, sl]
                return carry
            lax.fori_loop(0, RCH, orow, 0)
            pltpu.sync_copy(buf1.at[pl.ds(0, RCH)], u_hbm.at[urows])

    return k(h2, src4, dst3)


def kernel(x, edge_index, W1, b1, W2, b2):
    N, _ = x.shape
    D_OUT = W2.shape[0]
    E = edge_index.shape[1]
    DC = D_OUT // NC
    K = 10
    alpha = 0.1

    ET = E // NS
    B = 160
    NB = ET // B

    W2s = W2.reshape(NC, DC, W2.shape[1])
    b2s = b2.reshape(NC, 1, DC)
    h2 = _mlp_tc(x, W1, b1, W2s, b2s, DC)

    src3 = edge_index[0].reshape(NS, NB, B)
    dst3 = edge_index[1].reshape(NS, NB, B)
    src4 = jnp.stack([src3, src3 + N])

    u = _appnp_sc(h2, src4, dst3, N=N, DC=DC, K=K, alpha=alpha)
    return jnp.concatenate([u[:N], u[N:]], axis=1)


# speedup vs baseline: 18.6492x; 1.1818x over previous
import functools

import jax
import jax.numpy as jnp
from jax import lax
from jax.experimental import pallas as pl
from jax.experimental.pallas import tpu as pltpu
from jax.experimental.pallas import tpu_sc as plsc

NC = 2
NS = 16
LANES = 16


def _rsqrt16(v):
    i = plsc.bitcast(v, jnp.int32)
    i = jnp.int32(0x5F3759DF) - lax.shift_right_logical(i, 1)
    y = plsc.bitcast(i, jnp.float32)
    for _ in range(3):
        y = y * (1.5 - 0.5 * v * y * y)
    return y


def _mlp_tc(x, W1, b1, W2s, b2s, DC):
    N, D_IN = x.shape
    D_HID = W1.shape[0]
    BN = 1000

    def body(x_ref, w1_ref, b1_ref, w2_ref, b2_ref, o_ref):
        h1 = lax.dot_general(x_ref[...], w1_ref[...],
                             (((1,), (1,)), ((), ())),
                             preferred_element_type=jnp.float32)
        h1 = jnp.maximum(h1 + b1_ref[...], 0.0)
        for cc in range(NC):
            o_ref[cc] = lax.dot_general(h1, w2_ref[cc],
                                        (((1,), (1,)), ((), ())),
                                        preferred_element_type=jnp.float32) + b2_ref[cc]

    return pl.pallas_call(
        body,
        grid=(N // BN,),
        in_specs=[
            pl.BlockSpec((BN, D_IN), lambda i: (i, 0)),
            pl.BlockSpec((D_HID, D_IN), lambda i: (0, 0)),
            pl.BlockSpec((1, D_HID), lambda i: (0, 0)),
            pl.BlockSpec((NC, DC, D_HID), lambda i: (0, 0, 0)),
            pl.BlockSpec((NC, 1, DC), lambda i: (0, 0, 0)),
        ],
        out_specs=pl.BlockSpec((NC, BN, DC), lambda i: (0, i, 0)),
        out_shape=jax.ShapeDtypeStruct((NC, N, DC), jnp.float32),
    )(x, W1, b1.reshape(1, D_HID), W2s, b2s)


def _appnp_sc(h2, src4, dst3, *, N, DC, K, alpha):
    _, _, NB, B = src4.shape
    RT = N // NS
    RCH = 125
    NQ = RT // RCH
    keep = 1.0 - alpha

    mesh = plsc.VectorSubcoreMesh(core_axis_name="c", subcore_axis_name="s")

    @functools.partial(
        pl.kernel,
        out_type=jax.ShapeDtypeStruct((NC * N, DC), jnp.float32),
        mesh=mesh,
        compiler_params=pltpu.CompilerParams(use_tc_tiling_on_sc=False,
                                             needs_layout_passes=False),
        scratch_types=[
            pltpu.VMEM_SHARED((N, DC), jnp.float32),
            pltpu.VMEM((NB, B), jnp.int32),
            pltpu.VMEM((NB, B), jnp.int32),
            pltpu.VMEM((2, B, DC), jnp.float32),
            pltpu.VMEM((RCH, DC), jnp.float32),
            pltpu.VMEM((RT, LANES), jnp.float32),
            pltpu.SMEM((RT,), jnp.float32),
            pltpu.SMEM((RT,), jnp.float32),
            pltpu.SemaphoreType.DMA,
            pltpu.SemaphoreType.DMA,
        ],
    )
    def k(h_hbm, src_hbm, dst_hbm, u_hbm,
          agg_sh, src_v, dst_v, rbuf, buf1, sd_v, d2_s, a2_s, gsem, ssem):
        c = lax.axis_index("c")
        s = lax.axis_index("s")
        r0 = s * RT
        u0r = c * N + r0

        pltpu.sync_copy(src_hbm.at[c, s], src_v)
        pltpu.sync_copy(dst_hbm.at[s], dst_v)

        def gather_start(j, p):
            pltpu.async_copy(u_hbm.at[src_v.at[j]], rbuf.at[p], gsem)

        def gather_wait():
            pltpu.make_async_copy(u_hbm.at[src_v.at[0]], rbuf.at[0], gsem).wait()

        def scatter_start(j, p):
            pltpu.async_copy(rbuf.at[p], agg_sh.at[dst_v.at[j]], ssem, add=True)

        def scatter_start_ones(j):
            pltpu.async_copy(rbuf.at[0], agg_sh.at[dst_v.at[j]], ssem, add=True)

        def scatter_wait():
            pltpu.make_async_copy(rbuf.at[0], agg_sh.at[dst_v.at[0]],
                                  ssem).wait()

        def ones_body(i, carry):
            for g in range(DC // LANES):
                rbuf[0, i, pl.ds(LANES * g, LANES)] = jnp.full(
                    (LANES,), 1.0, jnp.float32)
            return carry
        lax.fori_loop(0, B, ones_body, 0)

        for q in range(NQ):
            pltpu.sync_copy(rbuf.at[0, pl.ds(0, RCH)],
                            agg_sh.at[pl.ds(r0 + q * RCH, RCH)])
        plsc.subcore_barrier()

        def deg_body(j, carry):
            scatter_start_ones(j)

            @pl.when(j >= 4)
            def _():
                scatter_wait()
            return carry
        lax.fori_loop(0, NB, deg_body, 0)
        for _ in range(4):
            scatter_wait()
        plsc.subcore_barrier()

        for q in range(NQ):
            rows = pl.ds(r0 + q * RCH, RCH)
            pltpu.sync_copy(agg_sh.at[rows], buf1.at[pl.ds(0, RCH)])
            pltpu.sync_copy(h_hbm.at[c, rows], rbuf.at[0, pl.ds(0, RCH)])

            def prow(rr, carry):
                r = q * RCH + rr
                dg = buf1[rr, pl.ds(0, LANES)]
                y = _rsqrt16(dg)
                d2_s[r] = jnp.max(keep / dg)
                a2_s[r] = jnp.max(alpha * y)
                sd_v[r, :] = dg * y
                for g in range(DC // LANES):
                    sl = pl.ds(LANES * g, LANES)
                    buf1[rr, sl] = y * rbuf[0, rr, sl]
                return carry
            lax.fori_loop(0, RCH, prow, 0)
            pltpu.sync_copy(buf1.at[pl.ds(0, RCH)],
                            u_hbm.at[pl.ds(u0r + q * RCH, RCH)])
            pltpu.sync_copy(buf1.at[pl.ds(0, RCH)], agg_sh.at[rows])
        plsc.subcore_barrier()

        for _t in range(K):
            with jax.named_scope("edge_phase"):
                gather_start(0, 0)

                def ebody(jj, carry):
                    j0 = 2 * jj
                    gather_wait()

                    @pl.when(jj > 0)
                    def _():
                        scatter_wait()
                    gather_start(j0 + 1, 1)
                    scatter_start(j0, 0)
                    gather_wait()
                    scatter_wait()
                    gather_start(j0 + 2, 0)
                    scatter_start(j0 + 1, 1)
                    return carry
                lax.fori_loop(0, NB // 2 - 1, ebody, 0)
                gather_wait()
                scatter_wait()
                gather_start(NB - 1, 1)
                scatter_start(NB - 2, 0)
                gather_wait()
                scatter_wait()
                scatter_start(NB - 1, 1)
                scatter_wait()
                plsc.subcore_barrier()

            with jax.named_scope("dense_phase"):
                for q in range(NQ):
                    rows = pl.ds(r0 + q * RCH, RCH)
                    pltpu.sync_copy(agg_sh.at[rows], buf1.at[pl.ds(0, RCH)])
                    pltpu.sync_copy(h_hbm.at[c, rows], rbuf.at[0, pl.ds(0, RCH)])

                    def drow(rr, carry):
                        r = q * RCH + rr
                        d2 = jnp.full((LANES,), d2_s[r], jnp.float32)
                        a2 = jnp.full((LANES,), a2_s[r], jnp.float32)
                        for g in range(DC // LANES):
                            sl = pl.ds(LANES * g, LANES)
                            buf1[rr, sl] = d2 * buf1[rr, sl] + a2 * rbuf[0, rr, sl]
                        return carry
                    lax.fori_loop(0, RCH, drow, 0)
                    pltpu.sync_copy(buf1.at[pl.ds(0, RCH)],
                                    u_hbm.at[pl.ds(u0r + q * RCH, RCH)])
                    pltpu.sync_copy(buf1.at[pl.ds(0, RCH)], agg_sh.at[rows])
                plsc.subcore_barrier()

        for q in range(NQ):
            urows = pl.ds(u0r + q * RCH, RCH)
            pltpu.sync_copy(u_hbm.at[urows], buf1.at[pl.ds(0, RCH)])

            def orow(rr, carry):
                r = q * RCH + rr
                sd = sd_v[r, :]
                for g in range(DC // LANES):
                    sl = pl.ds(LANES * g, LANES)
                    buf1[rr, sl] = sd * buf1[rr, sl]
                return carry
            lax.fori_loop(0, RCH, orow, 0)
            pltpu.sync_copy(buf1.at[pl.ds(0, RCH)], u_hbm.at[urows])

    return k(h2, src4, dst3)


def kernel(x, edge_index, W1, b1, W2, b2):
    N, _ = x.shape
    D_OUT = W2.shape[0]
    E = edge_index.shape[1]
    DC = D_OUT // NC
    K = 10
    alpha = 0.1

    ET = E // NS
    B = 250
    NB = ET // B

    W2s = W2.reshape(NC, DC, W2.shape[1])
    b2s = b2.reshape(NC, 1, DC)
    h2 = _mlp_tc(x, W1, b1, W2s, b2s, DC)

    src3 = edge_index[0].reshape(NS, NB, B)
    dst3 = edge_index[1].reshape(NS, NB, B)
    src4 = jnp.stack([src3, src3 + N])

    u = _appnp_sc(h2, src4, dst3, N=N, DC=DC, K=K, alpha=alpha)
    return jnp.concatenate([u[:N], u[N:]], axis=1)


# speedup vs baseline: 18.8127x; 1.0088x over previous
import functools

import jax
import jax.numpy as jnp
from jax import lax
from jax.experimental import pallas as pl
from jax.experimental.pallas import tpu as pltpu
from jax.experimental.pallas import tpu_sc as plsc

NC = 2
NS = 16
LANES = 16


def _rsqrt16(v):
    i = plsc.bitcast(v, jnp.int32)
    i = jnp.int32(0x5F3759DF) - lax.shift_right_logical(i, 1)
    y = plsc.bitcast(i, jnp.float32)
    for _ in range(3):
        y = y * (1.5 - 0.5 * v * y * y)
    return y


def _mlp_tc(x, W1, b1, W2s, b2s, DC):
    N, D_IN = x.shape
    D_HID = W1.shape[0]
    BN = 1000

    def body(x_ref, w1_ref, b1_ref, w2_ref, b2_ref, o_ref):
        h1 = lax.dot_general(x_ref[...], w1_ref[...],
                             (((1,), (1,)), ((), ())),
                             preferred_element_type=jnp.float32)
        h1 = jnp.maximum(h1 + b1_ref[...], 0.0)
        for cc in range(NC):
            o_ref[cc] = lax.dot_general(h1, w2_ref[cc],
                                        (((1,), (1,)), ((), ())),
                                        preferred_element_type=jnp.float32) + b2_ref[cc]

    return pl.pallas_call(
        body,
        grid=(N // BN,),
        in_specs=[
            pl.BlockSpec((BN, D_IN), lambda i: (i, 0)),
            pl.BlockSpec((D_HID, D_IN), lambda i: (0, 0)),
            pl.BlockSpec((1, D_HID), lambda i: (0, 0)),
            pl.BlockSpec((NC, DC, D_HID), lambda i: (0, 0, 0)),
            pl.BlockSpec((NC, 1, DC), lambda i: (0, 0, 0)),
        ],
        out_specs=pl.BlockSpec((NC, BN, DC), lambda i: (0, i, 0)),
        out_shape=jax.ShapeDtypeStruct((NC, N, DC), jnp.float32),
    )(x, W1, b1.reshape(1, D_HID), W2s, b2s)


def _appnp_sc(h2, src4, dst3, *, N, DC, K, alpha):
    _, _, NB, B = src4.shape
    RT = N // NS
    RCH = 125
    NQ = RT // RCH
    keep = 1.0 - alpha

    mesh = plsc.VectorSubcoreMesh(core_axis_name="c", subcore_axis_name="s")

    @functools.partial(
        pl.kernel,
        out_type=jax.ShapeDtypeStruct((NC * N, DC), jnp.float32),
        mesh=mesh,
        compiler_params=pltpu.CompilerParams(use_tc_tiling_on_sc=False,
                                             needs_layout_passes=False),
        scratch_types=[
            pltpu.VMEM_SHARED((N, DC), jnp.float32),
            pltpu.VMEM((NB, B), jnp.int32),
            pltpu.VMEM((NB, B), jnp.int32),
            pltpu.VMEM((2, B, DC), jnp.float32),
            pltpu.VMEM((2, RCH, DC), jnp.float32),
            pltpu.SMEM((RT,), jnp.float32),
            pltpu.SMEM((RT,), jnp.float32),
            pltpu.SemaphoreType.DMA,
            pltpu.SemaphoreType.DMA,
        ],
    )
    def k(h_hbm, src_hbm, dst_hbm, u_hbm,
          agg_sh, src_v, dst_v, rbuf, buf1, d2_s, a2_s, gsem, ssem):
        c = lax.axis_index("c")
        s = lax.axis_index("s")
        r0 = s * RT
        u0r = c * N + r0

        pltpu.sync_copy(src_hbm.at[c, s], src_v)
        pltpu.sync_copy(dst_hbm.at[s], dst_v)

        def gather_start(j, p):
            pltpu.async_copy(u_hbm.at[src_v.at[j]], rbuf.at[p], gsem)

        def gather_wait():
            pltpu.make_async_copy(u_hbm.at[src_v.at[0]], rbuf.at[0], gsem).wait()

        def scatter_start(j, p):
            pltpu.async_copy(rbuf.at[p], agg_sh.at[dst_v.at[j]], ssem, add=True)

        def scatter_start_ones(j):
            pltpu.async_copy(rbuf.at[0], agg_sh.at[dst_v.at[j]], ssem, add=True)

        def scatter_wait():
            pltpu.make_async_copy(rbuf.at[0], agg_sh.at[dst_v.at[0]],
                                  ssem).wait()

        def ones_body(i, carry):
            for g in range(DC // LANES):
                rbuf[0, i, pl.ds(LANES * g, LANES)] = jnp.full(
                    (LANES,), 1.0, jnp.float32)
            return carry
        lax.fori_loop(0, B, ones_body, 0)

        for q in range(NQ):
            pltpu.sync_copy(rbuf.at[0, pl.ds(0, RCH)],
                            agg_sh.at[pl.ds(r0 + q * RCH, RCH)])
        plsc.subcore_barrier()

        def deg_body(j, carry):
            scatter_start_ones(j)

            @pl.when(j >= 4)
            def _():
                scatter_wait()
            return carry
        lax.fori_loop(0, NB, deg_body, 0)
        for _ in range(4):
            scatter_wait()
        plsc.subcore_barrier()

        for q in range(NQ):
            rows = pl.ds(r0 + q * RCH, RCH)
            pltpu.sync_copy(agg_sh.at[rows], buf1.at[0])
            pltpu.sync_copy(h_hbm.at[c, rows], rbuf.at[0, pl.ds(0, RCH)])

            def prow(rr, carry):
                r = q * RCH + rr
                dg = buf1[0, rr, pl.ds(0, LANES)]
                y = _rsqrt16(dg)
                d2_s[r] = jnp.max(keep / dg)
                a2_s[r] = jnp.max(alpha * y)
                for g in range(DC // LANES):
                    sl = pl.ds(LANES * g, LANES)
                    buf1[0, rr, sl] = y * rbuf[0, rr, sl]
                return carry
            lax.fori_loop(0, RCH, prow, 0)
            pltpu.sync_copy(buf1.at[0],
                            u_hbm.at[pl.ds(u0r + q * RCH, RCH)])
            pltpu.sync_copy(buf1.at[0], agg_sh.at[rows])
        plsc.subcore_barrier()

        for _t in range(K):
            with jax.named_scope("edge_phase"):
                gather_start(0, 0)

                def ebody(jj, carry):
                    j0 = 2 * jj
                    gather_wait()

                    @pl.when(jj > 0)
                    def _():
                        scatter_wait()
                    gather_start(j0 + 1, 1)
                    scatter_start(j0, 0)
                    gather_wait()
                    scatter_wait()
                    gather_start(j0 + 2, 0)
                    scatter_start(j0 + 1, 1)
                    return carry
                lax.fori_loop(0, NB // 2 - 1, ebody, 0)
                gather_wait()
                scatter_wait()
                gather_start(NB - 1, 1)
                scatter_start(NB - 2, 0)
                gather_wait()
                scatter_wait()
                scatter_start(NB - 1, 1)
                scatter_wait()
                plsc.subcore_barrier()

            last = _t == K - 1
            with jax.named_scope("dense_phase"):
                def dense_in(q, p):
                    rows = pl.ds(r0 + q * RCH, RCH)
                    pltpu.sync_copy(agg_sh.at[rows], buf1.at[p])
                    pltpu.sync_copy(h_hbm.at[c, rows],
                                    rbuf.at[p, pl.ds(0, RCH)])

                for q in range(NQ):
                    p = q % 2
                    rows = pl.ds(r0 + q * RCH, RCH)
                    dense_in(q, p)

                    if not last:
                        def drow(rr, carry):
                            r = q * RCH + rr
                            d2 = jnp.full((LANES,), d2_s[r], jnp.float32)
                            a2 = jnp.full((LANES,), a2_s[r], jnp.float32)
                            for g in range(DC // LANES):
                                sl = pl.ds(LANES * g, LANES)
                                buf1[p, rr, sl] = (d2 * buf1[p, rr, sl]
                                                   + a2 * rbuf[p, rr, sl])
                            return carry
                        lax.fori_loop(0, RCH, drow, 0)
                        pltpu.sync_copy(buf1.at[p],
                                        u_hbm.at[pl.ds(u0r + q * RCH, RCH)])
                        pltpu.sync_copy(buf1.at[p], agg_sh.at[rows])
                    else:
                        def orow(rr, carry):
                            r = q * RCH + rr
                            d2 = jnp.full((LANES,), d2_s[r], jnp.float32)
                            a2 = jnp.full((LANES,), a2_s[r], jnp.float32)
                            m1 = (keep / alpha) * a2
                            m2 = a2 / d2
                            for g in range(DC // LANES):
                                sl = pl.ds(LANES * g, LANES)
                                buf1[p, rr, sl] = m1 * (buf1[p, rr, sl]
                                                        + m2 * rbuf[p, rr, sl])
                            return carry
                        lax.fori_loop(0, RCH, orow, 0)
                        pltpu.sync_copy(buf1.at[p],
                                        u_hbm.at[pl.ds(u0r + q * RCH, RCH)])
                if not last:
                    plsc.subcore_barrier()

    return k(h2, src4, dst3)


def kernel(x, edge_index, W1, b1, W2, b2):
    N, _ = x.shape
    D_OUT = W2.shape[0]
    E = edge_index.shape[1]
    DC = D_OUT // NC
    K = 10
    alpha = 0.1

    ET = E // NS
    B = 250
    NB = ET // B

    W2s = W2.reshape(NC, DC, W2.shape[1])
    b2s = b2.reshape(NC, 1, DC)
    h2 = _mlp_tc(x, W1, b1, W2s, b2s, DC)

    src3 = edge_index[0].reshape(NS, NB, B)
    dst3 = edge_index[1].reshape(NS, NB, B)
    src4 = jnp.stack([src3, src3 + N])

    u = _appnp_sc(h2, src4, dst3, N=N, DC=DC, K=K, alpha=alpha)
    return jnp.concatenate([u[:N], u[N:]], axis=1)


# speedup vs baseline: 21.8473x; 1.1613x over previous
import functools

import jax
import jax.numpy as jnp
from jax import lax
from jax.experimental import pallas as pl
from jax.experimental.pallas import tpu as pltpu
from jax.experimental.pallas import tpu_sc as plsc

NC = 2
NS = 16
LANES = 16


def _rsqrt16(v):
    i = plsc.bitcast(v, jnp.int32)
    i = jnp.int32(0x5F3759DF) - lax.shift_right_logical(i, 1)
    y = plsc.bitcast(i, jnp.float32)
    for _ in range(3):
        y = y * (1.5 - 0.5 * v * y * y)
    return y


def _mlp_tc(x, W1, b1, W2s, b2s, DC):
    N, D_IN = x.shape
    D_HID = W1.shape[0]
    BN = 1000

    def body(x_ref, w1_ref, b1_ref, w2_ref, b2_ref, o_ref):
        h1 = lax.dot_general(x_ref[...], w1_ref[...],
                             (((1,), (1,)), ((), ())),
                             preferred_element_type=jnp.float32)
        h1 = jnp.maximum(h1 + b1_ref[...], 0.0)
        for cc in range(NC):
            o_ref[cc] = lax.dot_general(h1, w2_ref[cc],
                                        (((1,), (1,)), ((), ())),
                                        preferred_element_type=jnp.float32) + b2_ref[cc]

    return pl.pallas_call(
        body,
        grid=(N // BN,),
        in_specs=[
            pl.BlockSpec((BN, D_IN), lambda i: (i, 0)),
            pl.BlockSpec((D_HID, D_IN), lambda i: (0, 0)),
            pl.BlockSpec((1, D_HID), lambda i: (0, 0)),
            pl.BlockSpec((NC, DC, D_HID), lambda i: (0, 0, 0)),
            pl.BlockSpec((NC, 1, DC), lambda i: (0, 0, 0)),
        ],
        out_specs=pl.BlockSpec((NC, BN, DC), lambda i: (0, i, 0)),
        out_shape=jax.ShapeDtypeStruct((NC, N, DC), jnp.float32),
    )(x, W1, b1.reshape(1, D_HID), W2s, b2s)


def _appnp_sc(h2, src4, dst3, *, N, DC, K, alpha):
    _, _, NB, B = src4.shape
    RT = N // NS
    RCH = 125
    NQ = RT // RCH
    keep = 1.0 - alpha

    mesh = plsc.VectorSubcoreMesh(core_axis_name="c", subcore_axis_name="s")

    @functools.partial(
        pl.kernel,
        out_type=jax.ShapeDtypeStruct((NC * N, DC), jnp.float32),
        mesh=mesh,
        compiler_params=pltpu.CompilerParams(use_tc_tiling_on_sc=False,
                                             needs_layout_passes=False),
        scratch_types=[
            pltpu.VMEM_SHARED((N, DC), jnp.float32),
            pltpu.VMEM((NB, B), jnp.int32),
            pltpu.VMEM((NB, B), jnp.int32),
            pltpu.VMEM((2, B, DC), jnp.float32),
            pltpu.VMEM((2, RCH, DC), jnp.float32),
            pltpu.SMEM((RT,), jnp.float32),
            pltpu.SMEM((RT,), jnp.float32),
            pltpu.SemaphoreType.DMA,
            pltpu.SemaphoreType.DMA,
        ],
    )
    def k(h_hbm, src_hbm, dst_hbm, u_hbm,
          agg_sh, src_v, dst_v, rbuf, buf1, d2_s, a2_s, gsem, ssem):
        c = lax.axis_index("c")
        s = lax.axis_index("s")
        r0 = s * RT
        u0r = c * N + r0

        pltpu.sync_copy(src_hbm.at[c, s], src_v)
        pltpu.sync_copy(dst_hbm.at[s], dst_v)

        def gather_start(j, p):
            pltpu.async_copy(u_hbm.at[src_v.at[j]], rbuf.at[p], gsem)

        def gather_wait():
            pltpu.make_async_copy(u_hbm.at[src_v.at[0]], rbuf.at[0], gsem).wait()

        def scatter_start(j, p):
            pltpu.async_copy(rbuf.at[p], agg_sh.at[dst_v.at[j]], ssem, add=True)

        def scatter_start_ones(j):
            pltpu.async_copy(rbuf.at[0], agg_sh.at[dst_v.at[j]], ssem, add=True)

        def scatter_wait():
            pltpu.make_async_copy(rbuf.at[0], agg_sh.at[dst_v.at[0]],
                                  ssem).wait()

        def ones_body(i, carry):
            for g in range(DC // LANES):
                rbuf[0, i, pl.ds(LANES * g, LANES)] = jnp.full(
                    (LANES,), 1.0, jnp.float32)
            return carry
        lax.fori_loop(0, B, ones_body, 0)

        for q in range(NQ):
            pltpu.sync_copy(rbuf.at[0, pl.ds(0, RCH)],
                            agg_sh.at[pl.ds(r0 + q * RCH, RCH)])
        plsc.subcore_barrier()

        def deg_body(j, carry):
            scatter_start_ones(j)

            @pl.when(j >= 4)
            def _():
                scatter_wait()
            return carry
        lax.fori_loop(0, NB, deg_body, 0)
        for _ in range(4):
            scatter_wait()
        plsc.subcore_barrier()

        for q in range(NQ):
            rows = pl.ds(r0 + q * RCH, RCH)
            pltpu.sync_copy(agg_sh.at[rows], buf1.at[0])
            pltpu.sync_copy(h_hbm.at[c, rows], rbuf.at[0, pl.ds(0, RCH)])

            def prow(rr, carry):
                r = q * RCH + rr
                dg = buf1[0, rr, pl.ds(0, LANES)]
                y = _rsqrt16(dg)
                d2_s[r] = jnp.max(keep / dg)
                a2_s[r] = jnp.max(alpha * y)
                for g in range(DC // LANES):
                    sl = pl.ds(LANES * g, LANES)
                    buf1[0, rr, sl] = y * rbuf[0, rr, sl]
                return carry
            lax.fori_loop(0, RCH, prow, 0)
            pltpu.sync_copy(buf1.at[0],
                            u_hbm.at[pl.ds(u0r + q * RCH, RCH)])
            pltpu.sync_copy(buf1.at[0], agg_sh.at[rows])
        plsc.subcore_barrier()

        for _t in range(K):
            with jax.named_scope("edge_phase"):
                gather_start(0, 0)
                gather_start(1, 1)

                def ebody(jj, carry):
                    j0 = 2 * jj
                    gather_wait()
                    scatter_start(j0, 0)
                    scatter_wait()
                    gather_start(j0 + 2, 0)
                    gather_wait()
                    scatter_start(j0 + 1, 1)
                    scatter_wait()
                    gather_start(j0 + 3, 1)
                    return carry
                lax.fori_loop(0, NB // 2 - 1, ebody, 0)
                gather_wait()
                scatter_start(NB - 2, 0)
                scatter_wait()
                gather_wait()
                scatter_start(NB - 1, 1)
                scatter_wait()
                plsc.subcore_barrier()

            last = _t == K - 1
            with jax.named_scope("dense_phase"):
                def dense_in(q, p):
                    rows = pl.ds(r0 + q * RCH, RCH)
                    pltpu.sync_copy(agg_sh.at[rows], buf1.at[p])
                    pltpu.sync_copy(h_hbm.at[c, rows],
                                    rbuf.at[p, pl.ds(0, RCH)])

                for q in range(NQ):
                    p = q % 2
                    rows = pl.ds(r0 + q * RCH, RCH)
                    dense_in(q, p)

                    if not last:
                        def drow(rr, carry):
                            r = q * RCH + rr
                            d2 = jnp.full((LANES,), d2_s[r], jnp.float32)
                            a2 = jnp.full((LANES,), a2_s[r], jnp.float32)
                            for g in range(DC // LANES):
                                sl = pl.ds(LANES * g, LANES)
                                buf1[p, rr, sl] = (d2 * buf1[p, rr, sl]
                                                   + a2 * rbuf[p, rr, sl])
                            return carry
                        lax.fori_loop(0, RCH, drow, 0)
                        pltpu.sync_copy(buf1.at[p],
                                        u_hbm.at[pl.ds(u0r + q * RCH, RCH)])
                        pltpu.sync_copy(buf1.at[p], agg_sh.at[rows])
                    else:
                        def orow(rr, carry):
                            r = q * RCH + rr
                            d2 = jnp.full((LANES,), d2_s[r], jnp.float32)
                            a2 = jnp.full((LANES,), a2_s[r], jnp.float32)
                            m1 = (keep / alpha) * a2
                            m2 = a2 / d2
                            for g in range(DC // LANES):
                                sl = pl.ds(LANES * g, LANES)
                                buf1[p, rr, sl] = m1 * (buf1[p, rr, sl]
                                                        + m2 * rbuf[p, rr, sl])
                            return carry
                        lax.fori_loop(0, RCH, orow, 0)
                        pltpu.sync_copy(buf1.at[p],
                                        u_hbm.at[pl.ds(u0r + q * RCH, RCH)])
                if not last:
                    plsc.subcore_barrier()

    return k(h2, src4, dst3)


def kernel(x, edge_index, W1, b1, W2, b2):
    N, _ = x.shape
    D_OUT = W2.shape[0]
    E = edge_index.shape[1]
    DC = D_OUT // NC
    K = 10
    alpha = 0.1

    ET = E // NS
    B = 250
    NB = ET // B

    W2s = W2.reshape(NC, DC, W2.shape[1])
    b2s = b2.reshape(NC, 1, DC)
    h2 = _mlp_tc(x, W1, b1, W2s, b2s, DC)

    src3 = edge_index[0].reshape(NS, NB, B)
    dst3 = edge_index[1].reshape(NS, NB, B)
    src4 = jnp.stack([src3, src3 + N])

    u = _appnp_sc(h2, src4, dst3, N=N, DC=DC, K=K, alpha=alpha)
    return jnp.concatenate([u[:N], u[N:]], axis=1)


# speedup vs baseline: 22.5927x; 1.0341x over previous
import functools

import jax
import jax.numpy as jnp
from jax import lax
from jax.experimental import pallas as pl
from jax.experimental.pallas import tpu as pltpu
from jax.experimental.pallas import tpu_sc as plsc

NC = 2
NS = 16
LANES = 16


def _rsqrt16(v):
    i = plsc.bitcast(v, jnp.int32)
    i = jnp.int32(0x5F3759DF) - lax.shift_right_logical(i, 1)
    y = plsc.bitcast(i, jnp.float32)
    for _ in range(3):
        y = y * (1.5 - 0.5 * v * y * y)
    return y


def _mlp_tc(x, W1, b1, W2s, b2s, DC):
    N, D_IN = x.shape
    D_HID = W1.shape[0]
    BN = 1000

    def body(x_ref, w1_ref, b1_ref, w2_ref, b2_ref, o_ref):
        h1 = lax.dot_general(x_ref[...], w1_ref[...],
                             (((1,), (1,)), ((), ())),
                             preferred_element_type=jnp.float32)
        h1 = jnp.maximum(h1 + b1_ref[...], 0.0)
        for cc in range(NC):
            o_ref[cc] = lax.dot_general(h1, w2_ref[cc],
                                        (((1,), (1,)), ((), ())),
                                        preferred_element_type=jnp.float32) + b2_ref[cc]

    return pl.pallas_call(
        body,
        grid=(N // BN,),
        in_specs=[
            pl.BlockSpec((BN, D_IN), lambda i: (i, 0)),
            pl.BlockSpec((D_HID, D_IN), lambda i: (0, 0)),
            pl.BlockSpec((1, D_HID), lambda i: (0, 0)),
            pl.BlockSpec((NC, DC, D_HID), lambda i: (0, 0, 0)),
            pl.BlockSpec((NC, 1, DC), lambda i: (0, 0, 0)),
        ],
        out_specs=pl.BlockSpec((NC, BN, DC), lambda i: (0, i, 0)),
        out_shape=jax.ShapeDtypeStruct((NC, N, DC), jnp.float32),
    )(x, W1, b1.reshape(1, D_HID), W2s, b2s)


def _appnp_sc(h2, src4, dst3, *, N, DC, K, alpha):
    _, _, NB, B = src4.shape
    RT = N // NS
    RCH = 125
    NQ = RT // RCH
    keep = 1.0 - alpha

    mesh = plsc.VectorSubcoreMesh(core_axis_name="c", subcore_axis_name="s")

    @functools.partial(
        pl.kernel,
        out_type=jax.ShapeDtypeStruct((NC * N, DC), jnp.float32),
        mesh=mesh,
        compiler_params=pltpu.CompilerParams(use_tc_tiling_on_sc=False,
                                             needs_layout_passes=False),
        scratch_types=[
            pltpu.VMEM_SHARED((N, DC), jnp.float32),
            pltpu.VMEM((NB, B), jnp.int32),
            pltpu.VMEM((NB, B), jnp.int32),
            pltpu.VMEM((2, B, DC), jnp.float32),
            pltpu.VMEM((2, RCH, DC), jnp.float32),
            pltpu.SMEM((RT,), jnp.float32),
            pltpu.SMEM((RT,), jnp.float32),
            pltpu.SemaphoreType.DMA,
            pltpu.SemaphoreType.DMA,
        ],
    )
    def k(h_hbm, src_hbm, dst_hbm, u_hbm,
          agg_sh, src_v, dst_v, rbuf, buf1, d2_s, a2_s, gsem, ssem):
        c = lax.axis_index("c")
        s = lax.axis_index("s")
        r0 = s * RT
        u0r = c * N + r0

        pltpu.sync_copy(src_hbm.at[c, s], src_v)
        pltpu.sync_copy(dst_hbm.at[s], dst_v)

        def gather_start(j, p):
            pltpu.async_copy(u_hbm.at[src_v.at[j]], rbuf.at[p], gsem)

        def gather_wait():
            pltpu.make_async_copy(u_hbm.at[src_v.at[0]], rbuf.at[0], gsem).wait()

        def scatter_start(j, p):
            pltpu.async_copy(rbuf.at[p], agg_sh.at[dst_v.at[j]], ssem, add=True)

        def scatter_start_ones(j):
            pltpu.async_copy(rbuf.at[0], agg_sh.at[dst_v.at[j]], ssem, add=True)

        def scatter_wait():
            pltpu.make_async_copy(rbuf.at[0], agg_sh.at[dst_v.at[0]],
                                  ssem).wait()

        def ones_body(i, carry):
            for g in range(DC // LANES):
                rbuf[0, i, pl.ds(LANES * g, LANES)] = jnp.full(
                    (LANES,), 1.0, jnp.float32)
            return carry
        lax.fori_loop(0, B, ones_body, 0)

        for q in range(NQ):
            pltpu.sync_copy(rbuf.at[0, pl.ds(0, RCH)],
                            agg_sh.at[pl.ds(r0 + q * RCH, RCH)])
        plsc.subcore_barrier()

        def deg_body(j, carry):
            scatter_start_ones(j)

            @pl.when(j >= 4)
            def _():
                scatter_wait()
            return carry
        lax.fori_loop(0, NB, deg_body, 0)
        for _ in range(4):
            scatter_wait()
        plsc.subcore_barrier()

        for q in range(NQ):
            rows = pl.ds(r0 + q * RCH, RCH)
            pltpu.sync_copy(agg_sh.at[rows], buf1.at[0])
            pltpu.sync_copy(h_hbm.at[c, rows], rbuf.at[0, pl.ds(0, RCH)])

            def prow(rr, carry):
                r = q * RCH + rr
                dg = buf1[0, rr, pl.ds(0, LANES)]
                y = _rsqrt16(dg)
                d2_s[r] = jnp.max(keep / dg)
                a2_s[r] = jnp.max(alpha * y)
                for g in range(DC // LANES):
                    sl = pl.ds(LANES * g, LANES)
                    buf1[0, rr, sl] = y * rbuf[0, rr, sl]
                return carry
            lax.fori_loop(0, RCH, prow, 0)
            pltpu.sync_copy(buf1.at[0],
                            u_hbm.at[pl.ds(u0r + q * RCH, RCH)])
            pltpu.sync_copy(buf1.at[0], agg_sh.at[rows])
        plsc.subcore_barrier()

        for _t in range(K):
            with jax.named_scope("edge_phase"):
                gather_start(0, 0)
                gather_start(1, 1)

                def ebody(jj, carry):
                    j0 = 2 * jj
                    gather_wait()
                    scatter_start(j0, 0)
                    scatter_wait()
                    gather_start(j0 + 2, 0)
                    gather_wait()
                    scatter_start(j0 + 1, 1)
                    scatter_wait()
                    gather_start(j0 + 3, 1)
                    return carry
                lax.fori_loop(0, NB // 2 - 1, ebody, 0)
                gather_wait()
                scatter_start(NB - 2, 0)
                scatter_wait()
                gather_wait()
                scatter_start(NB - 1, 1)
                scatter_wait()
                plsc.subcore_barrier()

            last = _t == K - 1
            with jax.named_scope("dense_phase"):
                def h_in(q, p):
                    rows = pl.ds(r0 + q * RCH, RCH)
                    pltpu.async_copy(h_hbm.at[c, rows],
                                     rbuf.at[p, pl.ds(0, RCH)], gsem)

                def h_wait():
                    pltpu.make_async_copy(h_hbm.at[c, pl.ds(r0, RCH)],
                                          rbuf.at[0, pl.ds(0, RCH)],
                                          gsem).wait()

                h_in(0, 0)
                for q in range(NQ):
                    p = q % 2
                    rows = pl.ds(r0 + q * RCH, RCH)
                    if q < NQ - 1:
                        h_in(q + 1, 1 - p)
                    pltpu.sync_copy(agg_sh.at[rows], buf1.at[p])
                    h_wait()

                    if not last:
                        def drow(rr, carry):
                            r = q * RCH + rr
                            d2 = jnp.full((LANES,), d2_s[r], jnp.float32)
                            a2 = jnp.full((LANES,), a2_s[r], jnp.float32)
                            for g in range(DC // LANES):
                                sl = pl.ds(LANES * g, LANES)
                                buf1[p, rr, sl] = (d2 * buf1[p, rr, sl]
                                                   + a2 * rbuf[p, rr, sl])
                            return carry
                        lax.fori_loop(0, RCH, drow, 0)
                        pltpu.sync_copy(buf1.at[p],
                                        u_hbm.at[pl.ds(u0r + q * RCH, RCH)])
                        pltpu.sync_copy(buf1.at[p], agg_sh.at[rows])
                    else:
                        def orow(rr, carry):
                            r = q * RCH + rr
                            d2 = jnp.full((LANES,), d2_s[r], jnp.float32)
                            a2 = jnp.full((LANES,), a2_s[r], jnp.float32)
                            m1 = (keep / alpha) * a2
                            m2 = a2 / d2
                            for g in range(DC // LANES):
                                sl = pl.ds(LANES * g, LANES)
                                buf1[p, rr, sl] = m1 * (buf1[p, rr, sl]
                                                        + m2 * rbuf[p, rr, sl])
                            return carry
                        lax.fori_loop(0, RCH, orow, 0)
                        pltpu.sync_copy(buf1.at[p],
                                        u_hbm.at[pl.ds(u0r + q * RCH, RCH)])
                if not last:
                    plsc.subcore_barrier()

    return k(h2, src4, dst3)


def kernel(x, edge_index, W1, b1, W2, b2):
    N, _ = x.shape
    D_OUT = W2.shape[0]
    E = edge_index.shape[1]
    DC = D_OUT // NC
    K = 10
    alpha = 0.1

    ET = E // NS
    B = 250
    NB = ET // B

    W2s = W2.reshape(NC, DC, W2.shape[1])
    b2s = b2.reshape(NC, 1, DC)
    h2 = _mlp_tc(x, W1, b1, W2s, b2s, DC)

    src3 = edge_index[0].reshape(NS, NB, B)
    dst3 = edge_index[1].reshape(NS, NB, B)
    src4 = jnp.stack([src3, src3 + N])

    u = _appnp_sc(h2, src4, dst3, N=N, DC=DC, K=K, alpha=alpha)
    return jnp.concatenate([u[:N], u[N:]], axis=1)


# speedup vs baseline: 24.5803x; 1.0880x over previous
import functools

import jax
import jax.numpy as jnp
from jax import lax
from jax.experimental import pallas as pl
from jax.experimental.pallas import tpu as pltpu
from jax.experimental.pallas import tpu_sc as plsc

NC = 2
NS = 16
LANES = 16


def _rsqrt16(v):
    i = plsc.bitcast(v, jnp.int32)
    i = jnp.int32(0x5F3759DF) - lax.shift_right_logical(i, 1)
    y = plsc.bitcast(i, jnp.float32)
    for _ in range(3):
        y = y * (1.5 - 0.5 * v * y * y)
    return y


def _mlp_tc(x, W1, b1, W2s, b2s, DC):
    N, D_IN = x.shape
    D_HID = W1.shape[0]
    BN = 1000

    def body(x_ref, w1_ref, b1_ref, w2_ref, b2_ref, o_ref):
        h1 = lax.dot_general(x_ref[...], w1_ref[...],
                             (((1,), (1,)), ((), ())),
                             preferred_element_type=jnp.float32)
        h1 = jnp.maximum(h1 + b1_ref[...], 0.0)
        for cc in range(NC):
            o_ref[cc] = lax.dot_general(h1, w2_ref[cc],
                                        (((1,), (1,)), ((), ())),
                                        preferred_element_type=jnp.float32) + b2_ref[cc]

    return pl.pallas_call(
        body,
        grid=(N // BN,),
        in_specs=[
            pl.BlockSpec((BN, D_IN), lambda i: (i, 0)),
            pl.BlockSpec((D_HID, D_IN), lambda i: (0, 0)),
            pl.BlockSpec((1, D_HID), lambda i: (0, 0)),
            pl.BlockSpec((NC, DC, D_HID), lambda i: (0, 0, 0)),
            pl.BlockSpec((NC, 1, DC), lambda i: (0, 0, 0)),
        ],
        out_specs=pl.BlockSpec((NC, BN, DC), lambda i: (0, i, 0)),
        out_shape=jax.ShapeDtypeStruct((NC, N, DC), jnp.float32),
    )(x, W1, b1.reshape(1, D_HID), W2s, b2s)


def _appnp_sc(h2, src4, dst3, *, N, DC, K, alpha):
    _, _, NB, B = src4.shape
    RT = N // NS
    RCH = 125
    NQ = RT // RCH
    keep = 1.0 - alpha

    mesh = plsc.VectorSubcoreMesh(core_axis_name="c", subcore_axis_name="s")

    @functools.partial(
        pl.kernel,
        out_type=jax.ShapeDtypeStruct((NC * N, DC), jnp.float32),
        mesh=mesh,
        compiler_params=pltpu.CompilerParams(use_tc_tiling_on_sc=False,
                                             needs_layout_passes=False),
        scratch_types=[
            pltpu.VMEM_SHARED((N, DC), jnp.float32),
            pltpu.VMEM((NB, B), jnp.int32),
            pltpu.VMEM((NB, B), jnp.int32),
            pltpu.VMEM((3, B, DC), jnp.float32),
            pltpu.VMEM((RCH, DC), jnp.float32),
            pltpu.SMEM((RT,), jnp.float32),
            pltpu.SMEM((RT,), jnp.float32),
            pltpu.SemaphoreType.DMA,
            pltpu.SemaphoreType.DMA,
        ],
    )
    def k(h_hbm, src_hbm, dst_hbm, u_hbm,
          agg_sh, src_v, dst_v, rbuf, buf1, d2_s, a2_s, gsem, ssem):
        c = lax.axis_index("c")
        s = lax.axis_index("s")
        r0 = s * RT
        u0r = c * N + r0

        pltpu.sync_copy(src_hbm.at[c, s], src_v)
        pltpu.sync_copy(dst_hbm.at[s], dst_v)

        def gather_start(j, p):
            pltpu.async_copy(u_hbm.at[src_v.at[j]], rbuf.at[p], gsem)

        def gather_wait():
            pltpu.make_async_copy(u_hbm.at[src_v.at[0]], rbuf.at[0], gsem).wait()

        def scatter_start(j, p):
            pltpu.async_copy(rbuf.at[p], agg_sh.at[dst_v.at[j]], ssem, add=True)

        def scatter_start_ones(j):
            pltpu.async_copy(rbuf.at[0], agg_sh.at[dst_v.at[j]], ssem, add=True)

        def scatter_wait():
            pltpu.make_async_copy(rbuf.at[0], agg_sh.at[dst_v.at[0]],
                                  ssem).wait()

        def ones_body(i, carry):
            for g in range(DC // LANES):
                rbuf[0, i, pl.ds(LANES * g, LANES)] = jnp.full(
                    (LANES,), 1.0, jnp.float32)
            return carry
        lax.fori_loop(0, B, ones_body, 0)

        for q in range(NQ):
            pltpu.sync_copy(rbuf.at[0, pl.ds(0, RCH)],
                            agg_sh.at[pl.ds(r0 + q * RCH, RCH)])
        plsc.subcore_barrier()

        def deg_body(j, carry):
            scatter_start_ones(j)

            @pl.when(j >= 4)
            def _():
                scatter_wait()
            return carry
        lax.fori_loop(0, NB, deg_body, 0)
        for _ in range(4):
            scatter_wait()
        plsc.subcore_barrier()

        for q in range(NQ):
            rows = pl.ds(r0 + q * RCH, RCH)
            pltpu.sync_copy(agg_sh.at[rows], buf1)
            pltpu.sync_copy(h_hbm.at[c, rows], rbuf.at[0, pl.ds(0, RCH)])

            def prow(rr, carry):
                r = q * RCH + rr
                dg = buf1[rr, pl.ds(0, LANES)]
                y = _rsqrt16(dg)
                d2_s[r] = jnp.max(keep / dg)
                a2_s[r] = jnp.max(alpha * y)
                for g in range(DC // LANES):
                    sl = pl.ds(LANES * g, LANES)
                    buf1[rr, sl] = y * rbuf[0, rr, sl]
                return carry
            lax.fori_loop(0, RCH, prow, 0)
            pltpu.sync_copy(buf1,
                            u_hbm.at[pl.ds(u0r + q * RCH, RCH)])
            pltpu.sync_copy(buf1, agg_sh.at[rows])
        plsc.subcore_barrier()

        for _t in range(K):
            with jax.named_scope("edge_phase"):
                gather_start(0, 0)
                gather_start(1, 1)

                def ebody(m, carry):
                    for i in range(3):
                        j = 3 * m + i
                        pj = i % 3
                        gather_wait()
                        scatter_start(j, pj)
                        if i == 0:
                            @pl.when(m > 0)
                            def _():
                                scatter_wait()
                        else:
                            scatter_wait()
                        gather_start(j + 2, (pj + 2) % 3)
                    return carry
                lax.fori_loop(0, NB // 3 - 1, ebody, 0)
                for j in range(3 * (NB // 3 - 1), NB):
                    pj = j % 3
                    gather_wait()
                    scatter_start(j, pj)
                    scatter_wait()
                    if j + 2 < NB:
                        gather_start(j + 2, (j + 2) % 3)
                scatter_wait()
                plsc.subcore_barrier()

            last = _t == K - 1
            with jax.named_scope("dense_phase"):
                def h_in(q, p):
                    rows = pl.ds(r0 + q * RCH, RCH)
                    pltpu.async_copy(h_hbm.at[c, rows],
                                     rbuf.at[p, pl.ds(0, RCH)], gsem)

                def h_wait():
                    pltpu.make_async_copy(h_hbm.at[c, pl.ds(r0, RCH)],
                                          rbuf.at[0, pl.ds(0, RCH)],
                                          gsem).wait()

                h_in(0, 0)
                for q in range(NQ):
                    p = q % 2
                    rows = pl.ds(r0 + q * RCH, RCH)
                    if q < NQ - 1:
                        h_in(q + 1, 1 - p)
                    pltpu.sync_copy(agg_sh.at[rows], buf1)
                    h_wait()

                    if not last:
                        def drow(rr, carry):
                            r = q * RCH + rr
                            d2 = jnp.full((LANES,), d2_s[r], jnp.float32)
                            a2 = jnp.full((LANES,), a2_s[r], jnp.float32)
                            for g in range(DC // LANES):
                                sl = pl.ds(LANES * g, LANES)
                                buf1[rr, sl] = (d2 * buf1[rr, sl]
                                                + a2 * rbuf[p, rr, sl])
                            return carry
                        lax.fori_loop(0, RCH, drow, 0)
                        pltpu.sync_copy(buf1,
                                        u_hbm.at[pl.ds(u0r + q * RCH, RCH)])
                        pltpu.sync_copy(buf1, agg_sh.at[rows])
                    else:
                        def orow(rr, carry):
                            r = q * RCH + rr
                            d2 = jnp.full((LANES,), d2_s[r], jnp.float32)
                            a2 = jnp.full((LANES,), a2_s[r], jnp.float32)
                            m1 = (keep / alpha) * a2
                            m2 = a2 / d2
                            for g in range(DC // LANES):
                                sl = pl.ds(LANES * g, LANES)
                                buf1[rr, sl] = m1 * (buf1[rr, sl]
                                                     + m2 * rbuf[p, rr, sl])
                            return carry
                        lax.fori_loop(0, RCH, orow, 0)
                        pltpu.sync_copy(buf1,
                                        u_hbm.at[pl.ds(u0r + q * RCH, RCH)])
                if not last:
                    plsc.subcore_barrier()

    return k(h2, src4, dst3)


def kernel(x, edge_index, W1, b1, W2, b2):
    N, _ = x.shape
    D_OUT = W2.shape[0]
    E = edge_index.shape[1]
    DC = D_OUT // NC
    K = 10
    alpha = 0.1

    ET = E // NS
    B = 200
    NB = ET // B

    W2s = W2.reshape(NC, DC, W2.shape[1])
    b2s = b2.reshape(NC, 1, DC)
    h2 = _mlp_tc(x, W1, b1, W2s, b2s, DC)

    src3 = edge_index[0].reshape(NS, NB, B)
    dst3 = edge_index[1].reshape(NS, NB, B)
    src4 = jnp.stack([src3, src3 + N])

    u = _appnp_sc(h2, src4, dst3, N=N, DC=DC, K=K, alpha=alpha)
    return jnp.concatenate([u[:N], u[N:]], axis=1)
